# baseline (speedup 1.0000x reference)
import functools

import jax
import jax.numpy as jnp
from jax import lax
from jax.experimental import pallas as pl
from jax.experimental.pallas import tpu as pltpu
from jax.experimental.pallas import tpu_sc as plsc

T, N, E, D, H, AEH, LAT = 4, 10000, 320000, 128, 128, 64, 32

NC, NS = 2, 16
NW = NC * NS
CH = 128
NCHUNK = 80
EPT = NCHUNK * CH
E_PAD = NW * EPT
PADE = E_PAD - E
N_PAD = 10240
RPT = N_PAD // NS
WCH = RPT // CH
DH = D // 2

_mesh = plsc.VectorSubcoreMesh(core_axis_name="c", subcore_axis_name="s")


NBUF = 4
LEAD = NBUF // 2


def _sc_body(with_deg, args):
    args = list(args)
    h0_hbm, h1_hbm, src_hbm, dst_hbm, zz64_hbm = args[:5]
    pos = 5
    if with_deg:
        zz16_hbm = args[pos]
        pos += 1
    else:
        zz16_hbm = None
    out0_hbm, out1_hbm = args[pos:pos + 2]
    pos += 2
    if with_deg:
        deg_hbm = args[pos]
        pos += 1
    else:
        deg_hbm = None
    bufs = args[pos:pos + NBUF]
    pos += NBUF
    srcb, dstb = args[pos:pos + 2]
    pos += 2
    if with_deg:
        ones16 = args[pos]
        pos += 1
    else:
        ones16 = None
    acc = args[pos]
    pos += 1
    if with_deg:
        dacc = args[pos]
        pos += 1
    else:
        dacc = None
    gsems = args[pos:pos + NBUF]
    pos += NBUF
    ssems = args[pos:pos + NBUF]
    pos += NBUF
    dsems = args[pos:pos + NBUF] if with_deg else [None] * NBUF

    cid = lax.axis_index("c")
    sid = lax.axis_index("s")
    wid = cid * NS + sid
    base = sid * RPT
    slab = pl.ds(base, RPT)

    pltpu.async_copy(zz64_hbm.at[slab], acc.at[slab], gsems[0])
    pltpu.async_copy(src_hbm.at[wid], srcb, ssems[0])
    pltpu.async_copy(dst_hbm.at[wid], dstb, ssems[1])
    if with_deg:
        pltpu.async_copy(zz16_hbm.at[slab], dacc.at[slab], gsems[1])
        one16 = jnp.full((16,), 1.0, jnp.float32)

        @pl.loop(0, CH)
        def _(r):
            ones16[r, pl.ds(0, 16)] = one16

    pltpu.make_async_copy(zz64_hbm.at[slab], acc.at[slab], gsems[0]).wait()
    pltpu.make_async_copy(src_hbm.at[wid], srcb, ssems[0]).wait()
    pltpu.make_async_copy(dst_hbm.at[wid], dstb, ssems[1]).wait()
    if with_deg:
        pltpu.make_async_copy(zz16_hbm.at[slab], dacc.at[slab],
                              gsems[1]).wait()

    plsc.subcore_barrier()

    def edge_pass(h_hbm, do_deg):
        def wait_gather(b):
            pltpu.make_async_copy(h_hbm.at[pl.ds(0, CH)], bufs[b],
                                  gsems[b]).wait()

        def wait_scatter(b):
            pltpu.make_async_copy(h_hbm.at[pl.ds(0, CH)], bufs[b],
                                  ssems[b]).wait()
            if do_deg:
                pltpu.make_async_copy(deg_hbm.at[0].at[pl.ds(0, CH)],
                                      ones16, dsems[b]).wait()

        nproc = NCHUNK
        for j in range(LEAD):
            pltpu.async_copy(h_hbm.at[srcb.at[j]], bufs[j], gsems[j])

        @pl.loop(0, nproc // NBUF)
        def _(i):
            for j in range(NBUF):
                g = NBUF * i + j
                b2 = (j + LEAD) % NBUF
                wait_gather(j)
                pltpu.async_copy(bufs[j], acc.at[dstb.at[g]], ssems[j],
                                 add=True)
                if do_deg:
                    pltpu.async_copy(ones16, dacc.at[dstb.at[g]], dsems[j],
                                     add=True)

                @pl.when(g >= LEAD)
                def _():
                    wait_scatter(b2)

                @pl.when(g + LEAD < nproc)
                def _():
                    pltpu.async_copy(h_hbm.at[srcb.at[g + LEAD]], bufs[b2],
                                     gsems[b2])

        for k in range(LEAD):
            wait_scatter((nproc - LEAD + k) % NBUF)

    with jax.named_scope("sc_pass0"):
        edge_pass(h0_hbm, with_deg)
    plsc.subcore_barrier()

    pltpu.async_copy(acc.at[slab], out0_hbm.at[cid].at[slab], gsems[0])
    if with_deg:
        pltpu.async_copy(dacc.at[slab], deg_hbm.at[cid].at[slab], gsems[1])
    pltpu.make_async_copy(acc.at[slab], out0_hbm.at[cid].at[slab],
                          gsems[0]).wait()
    if with_deg:
        pltpu.make_async_copy(dacc.at[slab], deg_hbm.at[cid].at[slab],
                              gsems[1]).wait()

    pltpu.sync_copy(zz64_hbm.at[slab], acc.at[slab])
    plsc.subcore_barrier()

    with jax.named_scope("sc_pass1"):
        edge_pass(h1_hbm, False)
    plsc.subcore_barrier()

    with jax.named_scope("sc_wout1"):
        pltpu.sync_copy(acc.at[slab], out1_hbm.at[cid].at[slab])


def _make_sc_agg(with_deg):
    out_type = [jax.ShapeDtypeStruct((NC, N_PAD, DH), jnp.float32),
                jax.ShapeDtypeStruct((NC, N_PAD, DH), jnp.float32)]
    scratch = [pltpu.VMEM((CH, DH), jnp.float32)] * NBUF
    scratch += [
        pltpu.VMEM((NCHUNK, CH), jnp.int32),
        pltpu.VMEM((NCHUNK, CH), jnp.int32),
    ]
    if with_deg:
        out_type.append(jax.ShapeDtypeStruct((NC, N_PAD, 16), jnp.float32))
        scratch.append(pltpu.VMEM((CH, 16), jnp.float32))
    scratch.append(pltpu.VMEM_SHARED((N_PAD, DH), jnp.float32))
    if with_deg:
        scratch.append(pltpu.VMEM_SHARED((N_PAD, 16), jnp.float32))
    nsem = 3 * NBUF if with_deg else 2 * NBUF
    scratch += [pltpu.SemaphoreType.DMA] * nsem

    def body(*args):
        _sc_body(with_deg, args)

    return pl.kernel(
        body,
        out_type=tuple(out_type),
        mesh=_mesh,
        scratch_types=scratch,
        compiler_params=pltpu.CompilerParams(use_tc_tiling_on_sc=False),
    )


_sc_agg_deg = _make_sc_agg(True)
_sc_agg = _make_sc_agg(False)


BN = 2000


def _sage_post_body(p00, p10, p01, p11, dg0, dg1, h, wlT, bl, wrT, g, b, out):
    deg = dg0[...][:, :1] + dg1[...][:, :1]
    inv = 1.0 / jnp.maximum(deg, 1.0)
    agg = jnp.concatenate([p00[...] + p10[...], p01[...] + p11[...]],
                          axis=-1) * inv
    y = (jnp.dot(agg, wlT[...], preferred_element_type=jnp.float32)
         + jnp.dot(h[...], wrT[...], preferred_element_type=jnp.float32)
         + bl[...])
    y = jnp.maximum(y, 0.0)
    mu = jnp.mean(y, axis=-1, keepdims=True)
    var = jnp.mean((y - mu) ** 2, axis=-1, keepdims=True)
    out[...] = (y - mu) * lax.rsqrt(var + 1e-5) * g[...] + b[...]


def _tc_sage(p00, p10, p01, p11, dg0, dg1, h, wlT, bl, wrT, g, b):
    row = lambda i: (i, 0)
    full = lambda i: (0, 0)
    return pl.pallas_call(
        _sage_post_body,
        grid=(N // BN,),
        in_specs=[
            pl.BlockSpec((BN, DH), row),
            pl.BlockSpec((BN, DH), row),
            pl.BlockSpec((BN, DH), row),
            pl.BlockSpec((BN, DH), row),
            pl.BlockSpec((BN, 16), row),
            pl.BlockSpec((BN, 16), row),
            pl.BlockSpec((BN, D), row),
            pl.BlockSpec((D, D), full),
            pl.BlockSpec((1, D), full),
            pl.BlockSpec((D, D), full),
            pl.BlockSpec((1, D), full),
            pl.BlockSpec((1, D), full),
        ],
        out_specs=pl.BlockSpec((BN, D), row),
        out_shape=jax.ShapeDtypeStruct((N, D), jnp.float32),
    )(p00, p10, p01, p11, dg0, dg1, h, wlT, bl, wrT, g, b)


def _lstm_mlp_body(e0, e1, e2, e3, wi0, wh0, b0, wi1, wh1, b1,
                   we1, be1, we2, be2, wd1, bd1, wd2, bd2, hm_out, hr_out):
    def layer(xs, wiT, whT, bb):
        hprev = jnp.zeros((BN, H), jnp.float32)
        c = jnp.zeros((BN, H), jnp.float32)
        hs = []
        for xt in xs:
            gates = (jnp.dot(xt, wiT, preferred_element_type=jnp.float32)
                     + jnp.dot(hprev, whT, preferred_element_type=jnp.float32)
                     + bb)
            i = jax.nn.sigmoid(gates[:, 0:H])
            f = jax.nn.sigmoid(gates[:, H:2 * H])
            gg = jnp.tanh(gates[:, 2 * H:3 * H])
            o = jax.nn.sigmoid(gates[:, 3 * H:4 * H])
            c = f * c + i * gg
            hprev = o * jnp.tanh(c)
            hs.append(hprev)
        return hs

    xs0 = [e0[...], e1[...], e2[...], e3[...]]
    hs0 = layer(xs0, wi0[...], wh0[...], b0[...])
    hs1 = layer(hs0, wi1[...], wh1[...], b1[...])
    hm = hs1[-1]
    z = jnp.maximum(
        jnp.dot(hm, we1[...], preferred_element_type=jnp.float32) + be1[...],
        0.0)
    z = jnp.dot(z, we2[...], preferred_element_type=jnp.float32) + be2[...]
    hr = jnp.maximum(
        jnp.dot(z, wd1[...], preferred_element_type=jnp.float32) + bd1[...],
        0.0)
    hr = jnp.dot(hr, wd2[...], preferred_element_type=jnp.float32) + bd2[...]
    hm_out[...] = hm
    hr_out[...] = hr


def _tc_lstm_mlp(e0, e1, e2, e3, wi0, wh0, b0, wi1, wh1, b1,
                 we1, be1, we2, be2, wd1, bd1, wd2, bd2):
    row = lambda i: (i, 0)
    full = lambda i: (0, 0)
    ws = [
        pl.BlockSpec((D, 4 * H), full),
        pl.BlockSpec((H, 4 * H), full),
        pl.BlockSpec((1, 4 * H), full),
        pl.BlockSpec((H, 4 * H), full),
        pl.BlockSpec((H, 4 * H), full),
        pl.BlockSpec((1, 4 * H), full),
        pl.BlockSpec((H, AEH), full),
        pl.BlockSpec((1, AEH), full),
        pl.BlockSpec((AEH, LAT), full),
        pl.BlockSpec((1, LAT), full),
        pl.BlockSpec((LAT, AEH), full),
        pl.BlockSpec((1, AEH), full),
        pl.BlockSpec((AEH, H), full),
        pl.BlockSpec((1, H), full),
    ]
    return pl.pallas_call(
        _lstm_mlp_body,
        grid=(N // BN,),
        in_specs=[pl.BlockSpec((BN, D), row)] * 4 + ws,
        out_specs=(pl.BlockSpec((BN, H), row), pl.BlockSpec((BN, H), row)),
        out_shape=(jax.ShapeDtypeStruct((N, H), jnp.float32),
                   jax.ShapeDtypeStruct((N, H), jnp.float32)),
    )(e0, e1, e2, e3, wi0, wh0, b0, wi1, wh1, b1,
      we1, be1, we2, be2, wd1, bd1, wd2, bd2)


def kernel(x, edge_index, Wl1, bl1, Wr1, ln1g, ln1b, Wl2, bl2, Wr2, ln2g,
           ln2b, Wih0, Whh0, bih0, bhh0, Wih1, Whh1, bih1, bhh1, We1, be1,
           We2, be2, Wd1, bd1, Wd2, bd2):
    src = edge_index[:, 0, :]
    dst = edge_index[:, 1, :]
    pad_i = jnp.arange(PADE, dtype=jnp.int32)
    pad_src = jnp.broadcast_to(pad_i % N, (T, PADE))
    pad_dst = jnp.broadcast_to(N + pad_i % (N_PAD - N), (T, PADE))
    srcp = jnp.concatenate([src, pad_src], axis=1).reshape(T, NW, NCHUNK, CH)
    dstp = jnp.concatenate([dst, pad_dst], axis=1).reshape(T, NW, NCHUNK, CH)

    r2 = lambda v: v.reshape(1, -1)
    Wl1T, Wr1T, Wl2T, Wr2T = Wl1.T, Wr1.T, Wl2.T, Wr2.T
    b0 = r2(bih0 + bhh0)
    b1 = r2(bih1 + bhh1)

    zz64 = jnp.zeros((N_PAD, DH), jnp.float32)
    zz16 = jnp.zeros((N_PAD, 16), jnp.float32)

    l1 = []
    for t in range(T):
        ht = x[t]
        ht0 = lax.slice_in_dim(ht, 0, DH, axis=1)
        ht1 = lax.slice_in_dim(ht, DH, D, axis=1)
        l1.append(_sc_agg_deg(ht0, ht1, srcp[t], dstp[t], zz64, zz16))

    embeds = []
    for t in range(T):
        ht = x[t]
        p0, p1, dgp = l1[t]
        dg0, dg1 = dgp[0, :N], dgp[1, :N]
        h1 = _tc_sage(p0[0, :N], p0[1, :N], p1[0, :N], p1[1, :N], dg0, dg1,
                      ht, Wl1T, r2(bl1), Wr1T, r2(ln1g), r2(ln1b))
        h10 = lax.slice_in_dim(h1, 0, DH, axis=1)
        h11 = lax.slice_in_dim(h1, DH, D, axis=1)
        q0, q1 = _sc_agg(h10, h11, srcp[t], dstp[t], zz64)
        h2 = _tc_sage(q0[0, :N], q0[1, :N], q1[0, :N], q1[1, :N], dg0, dg1,
                      h1, Wl2T, r2(bl2), Wr2T, r2(ln2g), r2(ln2b))
        embeds.append(h2)

    return _tc_lstm_mlp(
        embeds[0], embeds[1], embeds[2], embeds[3],
        Wih0.T, Whh0.T, b0, Wih1.T, Whh1.T, b1,
        We1.T, r2(be1), We2.T, r2(be2), Wd1.T, r2(bd1), Wd2.T, r2(bd2))


# speedup vs baseline: 1.0373x; 1.0373x over previous
import functools

import jax
import jax.numpy as jnp
from jax import lax
from jax.experimental import pallas as pl
from jax.experimental.pallas import tpu as pltpu
from jax.experimental.pallas import tpu_sc as plsc

T, N, E, D, H, AEH, LAT = 4, 10000, 320000, 128, 128, 64, 32

NC, NS = 2, 16
NW = NC * NS
CH = 128
NCHUNK = 80
EPT = NCHUNK * CH
E_PAD = NW * EPT
PADE = E_PAD - E
N_PAD = 10240
RPT = N_PAD // NS
WCH = RPT // CH
DH = D // 2

_mesh = plsc.VectorSubcoreMesh(core_axis_name="c", subcore_axis_name="s")


NBUF = 4
LEAD = NBUF // 2


def _sc_body(with_deg, args):
    args = list(args)
    h_hbm, src0_hbm, src1_hbm, dst_hbm, zz64_hbm = args[:5]
    pos = 5
    if with_deg:
        zz16_hbm = args[pos]
        pos += 1
    else:
        zz16_hbm = None
    out0_hbm, out1_hbm = args[pos:pos + 2]
    pos += 2
    if with_deg:
        deg_hbm = args[pos]
        pos += 1
    else:
        deg_hbm = None
    bufs = args[pos:pos + NBUF]
    pos += NBUF
    srcb0, srcb1, dstb = args[pos:pos + 3]
    pos += 3
    if with_deg:
        ones16 = args[pos]
        pos += 1
    else:
        ones16 = None
    acc = args[pos]
    pos += 1
    if with_deg:
        dacc = args[pos]
        pos += 1
    else:
        dacc = None
    gsems = args[pos:pos + NBUF]
    pos += NBUF
    ssems = args[pos:pos + NBUF]
    pos += NBUF
    dsems = args[pos:pos + NBUF] if with_deg else [None] * NBUF

    cid = lax.axis_index("c")
    sid = lax.axis_index("s")
    wid = cid * NS + sid
    base = sid * RPT
    slab = pl.ds(base, RPT)

    pltpu.async_copy(zz64_hbm.at[slab], acc.at[slab], gsems[0])
    pltpu.async_copy(src0_hbm.at[wid], srcb0, ssems[0])
    pltpu.async_copy(src1_hbm.at[wid], srcb1, ssems[2])
    pltpu.async_copy(dst_hbm.at[wid], dstb, ssems[1])
    if with_deg:
        pltpu.async_copy(zz16_hbm.at[slab], dacc.at[slab], gsems[1])
        one16 = jnp.full((16,), 1.0, jnp.float32)

        @pl.loop(0, CH)
        def _(r):
            ones16[r, pl.ds(0, 16)] = one16

    pltpu.make_async_copy(zz64_hbm.at[slab], acc.at[slab], gsems[0]).wait()
    pltpu.make_async_copy(src0_hbm.at[wid], srcb0, ssems[0]).wait()
    pltpu.make_async_copy(src1_hbm.at[wid], srcb1, ssems[2]).wait()
    pltpu.make_async_copy(dst_hbm.at[wid], dstb, ssems[1]).wait()
    if with_deg:
        pltpu.make_async_copy(zz16_hbm.at[slab], dacc.at[slab],
                              gsems[1]).wait()

    plsc.subcore_barrier()

    def edge_pass(srcb, do_deg):
        def wait_gather(b):
            pltpu.make_async_copy(h_hbm.at[pl.ds(0, CH)], bufs[b],
                                  gsems[b]).wait()

        def wait_scatter(b):
            pltpu.make_async_copy(h_hbm.at[pl.ds(0, CH)], bufs[b],
                                  ssems[b]).wait()
            if do_deg:
                pltpu.make_async_copy(deg_hbm.at[0].at[pl.ds(0, CH)],
                                      ones16, dsems[b]).wait()

        nproc = NCHUNK
        for j in range(LEAD):
            pltpu.async_copy(h_hbm.at[srcb.at[j]], bufs[j], gsems[j])

        @pl.loop(0, nproc // NBUF)
        def _(i):
            for j in range(NBUF):
                g = NBUF * i + j
                b2 = (j + LEAD) % NBUF
                wait_gather(j)
                pltpu.async_copy(bufs[j], acc.at[dstb.at[g]], ssems[j],
                                 add=True)
                if do_deg:
                    pltpu.async_copy(ones16, dacc.at[dstb.at[g]], dsems[j],
                                     add=True)

                @pl.when(g >= LEAD)
                def _():
                    wait_scatter(b2)

                @pl.when(g + LEAD < nproc)
                def _():
                    pltpu.async_copy(h_hbm.at[srcb.at[g + LEAD]], bufs[b2],
                                     gsems[b2])

        for k in range(LEAD):
            wait_scatter((nproc - LEAD + k) % NBUF)

    with jax.named_scope("sc_pass0"):
        edge_pass(srcb0, with_deg)
    plsc.subcore_barrier()

    pltpu.async_copy(acc.at[slab], out0_hbm.at[cid].at[slab], gsems[0])
    if with_deg:
        pltpu.async_copy(dacc.at[slab], deg_hbm.at[cid].at[slab], gsems[1])
    pltpu.make_async_copy(acc.at[slab], out0_hbm.at[cid].at[slab],
                          gsems[0]).wait()
    if with_deg:
        pltpu.make_async_copy(dacc.at[slab], deg_hbm.at[cid].at[slab],
                              gsems[1]).wait()

    pltpu.sync_copy(zz64_hbm.at[slab], acc.at[slab])
    plsc.subcore_barrier()

    with jax.named_scope("sc_pass1"):
        edge_pass(srcb1, False)
    plsc.subcore_barrier()

    with jax.named_scope("sc_wout1"):
        pltpu.sync_copy(acc.at[slab], out1_hbm.at[cid].at[slab])


def _make_sc_agg(with_deg):
    out_type = [jax.ShapeDtypeStruct((NC, N_PAD, DH), jnp.float32),
                jax.ShapeDtypeStruct((NC, N_PAD, DH), jnp.float32)]
    scratch = [pltpu.VMEM((CH, DH), jnp.float32)] * NBUF
    scratch += [
        pltpu.VMEM((NCHUNK, CH), jnp.int32),
        pltpu.VMEM((NCHUNK, CH), jnp.int32),
        pltpu.VMEM((NCHUNK, CH), jnp.int32),
    ]
    if with_deg:
        out_type.append(jax.ShapeDtypeStruct((NC, N_PAD, 16), jnp.float32))
        scratch.append(pltpu.VMEM((CH, 16), jnp.float32))
    scratch.append(pltpu.VMEM_SHARED((N_PAD, DH), jnp.float32))
    if with_deg:
        scratch.append(pltpu.VMEM_SHARED((N_PAD, 16), jnp.float32))
    nsem = 3 * NBUF if with_deg else 2 * NBUF
    scratch += [pltpu.SemaphoreType.DMA] * nsem

    def body(*args):
        _sc_body(with_deg, args)

    return pl.kernel(
        body,
        out_type=tuple(out_type),
        mesh=_mesh,
        scratch_types=scratch,
        compiler_params=pltpu.CompilerParams(use_tc_tiling_on_sc=False),
    )


_sc_agg_deg = _make_sc_agg(True)
_sc_agg = _make_sc_agg(False)


BN = 2000


def _sage_post_body(p00, p10, p01, p11, dg0, dg1, h, wlT, bl, wrT, g, b, out):
    deg = dg0[...][:, :1] + dg1[...][:, :1]
    inv = 1.0 / jnp.maximum(deg, 1.0)
    agg = jnp.concatenate([p00[...] + p10[...], p01[...] + p11[...]],
                          axis=-1) * inv
    y = (jnp.dot(agg, wlT[...], preferred_element_type=jnp.float32)
         + jnp.dot(h[...], wrT[...], preferred_element_type=jnp.float32)
         + bl[...])
    y = jnp.maximum(y, 0.0)
    mu = jnp.mean(y, axis=-1, keepdims=True)
    var = jnp.mean((y - mu) ** 2, axis=-1, keepdims=True)
    out[...] = (y - mu) * lax.rsqrt(var + 1e-5) * g[...] + b[...]


def _tc_sage(p00, p10, p01, p11, dg0, dg1, h, wlT, bl, wrT, g, b):
    row = lambda i: (i, 0)
    full = lambda i: (0, 0)
    return pl.pallas_call(
        _sage_post_body,
        grid=(N // BN,),
        in_specs=[
            pl.BlockSpec((BN, DH), row),
            pl.BlockSpec((BN, DH), row),
            pl.BlockSpec((BN, DH), row),
            pl.BlockSpec((BN, DH), row),
            pl.BlockSpec((BN, 16), row),
            pl.BlockSpec((BN, 16), row),
            pl.BlockSpec((BN, D), row),
            pl.BlockSpec((D, D), full),
            pl.BlockSpec((1, D), full),
            pl.BlockSpec((D, D), full),
            pl.BlockSpec((1, D), full),
            pl.BlockSpec((1, D), full),
        ],
        out_specs=pl.BlockSpec((BN, D), row),
        out_shape=jax.ShapeDtypeStruct((N, D), jnp.float32),
    )(p00, p10, p01, p11, dg0, dg1, h, wlT, bl, wrT, g, b)


def _lstm_mlp_body(e0, e1, e2, e3, wi0, wh0, b0, wi1, wh1, b1,
                   we1, be1, we2, be2, wd1, bd1, wd2, bd2, hm_out, hr_out):
    def layer(xs, wiT, whT, bb):
        hprev = jnp.zeros((BN, H), jnp.float32)
        c = jnp.zeros((BN, H), jnp.float32)
        hs = []
        for xt in xs:
            gates = (jnp.dot(xt, wiT, preferred_element_type=jnp.float32)
                     + jnp.dot(hprev, whT, preferred_element_type=jnp.float32)
                     + bb)
            i = jax.nn.sigmoid(gates[:, 0:H])
            f = jax.nn.sigmoid(gates[:, H:2 * H])
            gg = jnp.tanh(gates[:, 2 * H:3 * H])
            o = jax.nn.sigmoid(gates[:, 3 * H:4 * H])
            c = f * c + i * gg
            hprev = o * jnp.tanh(c)
            hs.append(hprev)
        return hs

    xs0 = [e0[...], e1[...], e2[...], e3[...]]
    hs0 = layer(xs0, wi0[...], wh0[...], b0[...])
    hs1 = layer(hs0, wi1[...], wh1[...], b1[...])
    hm = hs1[-1]
    z = jnp.maximum(
        jnp.dot(hm, we1[...], preferred_element_type=jnp.float32) + be1[...],
        0.0)
    z = jnp.dot(z, we2[...], preferred_element_type=jnp.float32) + be2[...]
    hr = jnp.maximum(
        jnp.dot(z, wd1[...], preferred_element_type=jnp.float32) + bd1[...],
        0.0)
    hr = jnp.dot(hr, wd2[...], preferred_element_type=jnp.float32) + bd2[...]
    hm_out[...] = hm
    hr_out[...] = hr


def _tc_lstm_mlp(e0, e1, e2, e3, wi0, wh0, b0, wi1, wh1, b1,
                 we1, be1, we2, be2, wd1, bd1, wd2, bd2):
    row = lambda i: (i, 0)
    full = lambda i: (0, 0)
    ws = [
        pl.BlockSpec((D, 4 * H), full),
        pl.BlockSpec((H, 4 * H), full),
        pl.BlockSpec((1, 4 * H), full),
        pl.BlockSpec((H, 4 * H), full),
        pl.BlockSpec((H, 4 * H), full),
        pl.BlockSpec((1, 4 * H), full),
        pl.BlockSpec((H, AEH), full),
        pl.BlockSpec((1, AEH), full),
        pl.BlockSpec((AEH, LAT), full),
        pl.BlockSpec((1, LAT), full),
        pl.BlockSpec((LAT, AEH), full),
        pl.BlockSpec((1, AEH), full),
        pl.BlockSpec((AEH, H), full),
        pl.BlockSpec((1, H), full),
    ]
    return pl.pallas_call(
        _lstm_mlp_body,
        grid=(N // BN,),
        in_specs=[pl.BlockSpec((BN, D), row)] * 4 + ws,
        out_specs=(pl.BlockSpec((BN, H), row), pl.BlockSpec((BN, H), row)),
        out_shape=(jax.ShapeDtypeStruct((N, H), jnp.float32),
                   jax.ShapeDtypeStruct((N, H), jnp.float32)),
    )(e0, e1, e2, e3, wi0, wh0, b0, wi1, wh1, b1,
      we1, be1, we2, be2, wd1, bd1, wd2, bd2)


def kernel(x, edge_index, Wl1, bl1, Wr1, ln1g, ln1b, Wl2, bl2, Wr2, ln2g,
           ln2b, Wih0, Whh0, bih0, bhh0, Wih1, Whh1, bih1, bhh1, We1, be1,
           We2, be2, Wd1, bd1, Wd2, bd2):
    src = edge_index[:, 0, :]
    dst = edge_index[:, 1, :]
    pad_i = jnp.arange(PADE, dtype=jnp.int32)
    pad_src = jnp.broadcast_to(pad_i % N, (T, PADE))
    pad_dst = jnp.broadcast_to(N + pad_i % (N_PAD - N), (T, PADE))
    src2 = 2 * jnp.concatenate([src, pad_src], axis=1)
    srcp0 = src2.reshape(T, NW, NCHUNK, CH)
    srcp1 = (src2 + 1).reshape(T, NW, NCHUNK, CH)
    dstp = jnp.concatenate([dst, pad_dst], axis=1).reshape(T, NW, NCHUNK, CH)

    r2 = lambda v: v.reshape(1, -1)
    Wl1T, Wr1T, Wl2T, Wr2T = Wl1.T, Wr1.T, Wl2.T, Wr2.T
    b0 = r2(bih0 + bhh0)
    b1 = r2(bih1 + bhh1)

    zz64 = jnp.zeros((N_PAD, DH), jnp.float32)
    zz16 = jnp.zeros((N_PAD, 16), jnp.float32)

    l1 = []
    for t in range(T):
        ht2 = x[t].reshape(2 * N, DH)
        l1.append(_sc_agg_deg(ht2, srcp0[t], srcp1[t], dstp[t], zz64, zz16))

    embeds = []
    for t in range(T):
        ht = x[t]
        p0, p1, dgp = l1[t]
        dg0, dg1 = dgp[0, :N], dgp[1, :N]
        h1 = _tc_sage(p0[0, :N], p0[1, :N], p1[0, :N], p1[1, :N], dg0, dg1,
                      ht, Wl1T, r2(bl1), Wr1T, r2(ln1g), r2(ln1b))
        q0, q1 = _sc_agg(h1.reshape(2 * N, DH), srcp0[t], srcp1[t], dstp[t],
                         zz64)
        h2 = _tc_sage(q0[0, :N], q0[1, :N], q1[0, :N], q1[1, :N], dg0, dg1,
                      h1, Wl2T, r2(bl2), Wr2T, r2(ln2g), r2(ln2b))
        embeds.append(h2)

    return _tc_lstm_mlp(
        embeds[0], embeds[1], embeds[2], embeds[3],
        Wih0.T, Whh0.T, b0, Wih1.T, Whh1.T, b1,
        We1.T, r2(be1), We2.T, r2(be2), Wd1.T, r2(bd1), Wd2.T, r2(bd2))


# speedup vs baseline: 1.1316x; 1.0910x over previous
import functools

import jax
import jax.numpy as jnp
from jax import lax
from jax.experimental import pallas as pl
from jax.experimental.pallas import tpu as pltpu
from jax.experimental.pallas import tpu_sc as plsc

T, N, E, D, H, AEH, LAT = 4, 10000, 320000, 128, 128, 64, 32

NC, NS = 2, 16
NW = NC * NS
CH = 128
NCHUNK = 80
EPT = NCHUNK * CH
E_PAD = NW * EPT
PADE = E_PAD - E
N_PAD = 10240
RPT = N_PAD // NS
WCH = RPT // CH
DH = D // 2

_mesh = plsc.VectorSubcoreMesh(core_axis_name="c", subcore_axis_name="s")


NBUF = 4
LEAD = NBUF // 2


def _sc_body(with_deg, args):
    args = list(args)
    h_hbm, src0_hbm, src1_hbm, dst_hbm, zz64_hbm = args[:5]
    pos = 5
    if with_deg:
        zz16_hbm = args[pos]
        pos += 1
    else:
        zz16_hbm = None
    out0_hbm, out1_hbm = args[pos:pos + 2]
    pos += 2
    if with_deg:
        deg_hbm = args[pos]
        pos += 1
    else:
        deg_hbm = None
    bufs = args[pos:pos + NBUF]
    pos += NBUF
    srcb0, srcb1, dstb = args[pos:pos + 3]
    pos += 3
    if with_deg:
        ones16 = args[pos]
        pos += 1
    else:
        ones16 = None
    acc = args[pos]
    pos += 1
    if with_deg:
        dacc = args[pos]
        pos += 1
    else:
        dacc = None
    gsems = args[pos:pos + NBUF]
    pos += NBUF
    ssems = args[pos:pos + NBUF]
    pos += NBUF
    dsems = args[pos:pos + NBUF] if with_deg else [None] * NBUF

    cid = lax.axis_index("c")
    sid = lax.axis_index("s")
    wid = cid * NS + sid
    base = sid * RPT
    slab = pl.ds(base, RPT)

    pltpu.async_copy(zz64_hbm.at[slab], acc.at[slab], gsems[0])
    pltpu.async_copy(src0_hbm.at[wid], srcb0, ssems[0])
    pltpu.async_copy(src1_hbm.at[wid], srcb1, ssems[2])
    pltpu.async_copy(dst_hbm.at[wid], dstb, ssems[1])
    if with_deg:
        pltpu.async_copy(zz16_hbm.at[slab], dacc.at[slab], gsems[1])
        one16 = jnp.full((16,), 1.0, jnp.float32)

        @pl.loop(0, CH)
        def _(r):
            ones16[r, pl.ds(0, 16)] = one16

    pltpu.make_async_copy(zz64_hbm.at[slab], acc.at[slab], gsems[0]).wait()
    pltpu.make_async_copy(src0_hbm.at[wid], srcb0, ssems[0]).wait()
    pltpu.make_async_copy(src1_hbm.at[wid], srcb1, ssems[2]).wait()
    pltpu.make_async_copy(dst_hbm.at[wid], dstb, ssems[1]).wait()
    if with_deg:
        pltpu.make_async_copy(zz16_hbm.at[slab], dacc.at[slab],
                              gsems[1]).wait()

    plsc.subcore_barrier()

    def edge_pass(srcb, do_deg):
        def wait_gather(b):
            pltpu.make_async_copy(h_hbm.at[pl.ds(0, CH)], bufs[b],
                                  gsems[b]).wait()

        def wait_scatter(b):
            pltpu.make_async_copy(h_hbm.at[pl.ds(0, CH)], bufs[b],
                                  ssems[b]).wait()
            if do_deg:
                pltpu.make_async_copy(deg_hbm.at[0].at[pl.ds(0, CH)],
                                      ones16, dsems[b]).wait()

        nproc = NCHUNK
        for j in range(LEAD):
            pltpu.async_copy(h_hbm.at[srcb.at[j]], bufs[j], gsems[j])

        @pl.loop(0, nproc // NBUF)
        def _(i):
            for j in range(NBUF):
                g = NBUF * i + j
                b2 = (j + LEAD) % NBUF
                wait_gather(j)
                pltpu.async_copy(bufs[j], acc.at[dstb.at[g]], ssems[j],
                                 add=True)
                if do_deg:
                    pltpu.async_copy(ones16, dacc.at[dstb.at[g]], dsems[j],
                                     add=True)

                @pl.when(g >= LEAD)
                def _():
                    wait_scatter(b2)

                @pl.when(g + LEAD < nproc)
                def _():
                    pltpu.async_copy(h_hbm.at[srcb.at[g + LEAD]], bufs[b2],
                                     gsems[b2])

        for k in range(LEAD):
            wait_scatter((nproc - LEAD + k) % NBUF)

    with jax.named_scope("sc_pass0"):
        edge_pass(srcb0, with_deg)
    plsc.subcore_barrier()

    pltpu.async_copy(acc.at[slab], out0_hbm.at[cid].at[slab], gsems[0])
    if with_deg:
        pltpu.async_copy(dacc.at[slab], deg_hbm.at[cid].at[slab], gsems[1])
    pltpu.make_async_copy(acc.at[slab], out0_hbm.at[cid].at[slab],
                          gsems[0]).wait()
    if with_deg:
        pltpu.make_async_copy(dacc.at[slab], deg_hbm.at[cid].at[slab],
                              gsems[1]).wait()

    pltpu.sync_copy(zz64_hbm.at[slab], acc.at[slab])
    plsc.subcore_barrier()

    with jax.named_scope("sc_pass1"):
        edge_pass(srcb1, False)
    plsc.subcore_barrier()

    with jax.named_scope("sc_wout1"):
        pltpu.sync_copy(acc.at[slab], out1_hbm.at[cid].at[slab])


def _make_sc_agg(with_deg):
    out_type = [jax.ShapeDtypeStruct((NC, N_PAD, DH), jnp.float32),
                jax.ShapeDtypeStruct((NC, N_PAD, DH), jnp.float32)]
    scratch = [pltpu.VMEM((CH, DH), jnp.float32)] * NBUF
    scratch += [
        pltpu.VMEM((NCHUNK, CH), jnp.int32),
        pltpu.VMEM((NCHUNK, CH), jnp.int32),
        pltpu.VMEM((NCHUNK, CH), jnp.int32),
    ]
    if with_deg:
        out_type.append(jax.ShapeDtypeStruct((NC, N_PAD, 16), jnp.float32))
        scratch.append(pltpu.VMEM((CH, 16), jnp.float32))
    scratch.append(pltpu.VMEM_SHARED((N_PAD, DH), jnp.float32))
    if with_deg:
        scratch.append(pltpu.VMEM_SHARED((N_PAD, 16), jnp.float32))
    nsem = 3 * NBUF if with_deg else 2 * NBUF
    scratch += [pltpu.SemaphoreType.DMA] * nsem

    def body(*args):
        _sc_body(with_deg, args)

    return pl.kernel(
        body,
        out_type=tuple(out_type),
        mesh=_mesh,
        scratch_types=scratch,
        compiler_params=pltpu.CompilerParams(use_tc_tiling_on_sc=False),
    )


_sc_agg_deg = _make_sc_agg(True)
_sc_agg = _make_sc_agg(False)


BN = 2000


def _sage_post_body(p0a, p0b, p1a, p1b, dg, hE, hO, wlT0, wlT1, bl, wrT,
                    g, b, out):
    p0 = p0a[...][0] + p0b[...][0]
    p1 = p1a[...][0] + p1b[...][0]
    dgv = dg[...]
    wl0 = wlT0[...]
    wl1 = wlT1[...]
    wr = wrT[...]

    def half(agg0, agg1, inv, hv):
        y = (jnp.dot(agg0 * inv, wl0, preferred_element_type=jnp.float32)
             + jnp.dot(agg1 * inv, wl1, preferred_element_type=jnp.float32)
             + jnp.dot(hv, wr, preferred_element_type=jnp.float32)
             + bl[...])
        y = jnp.maximum(y, 0.0)
        mu = jnp.mean(y, axis=-1, keepdims=True)
        var = jnp.mean((y - mu) ** 2, axis=-1, keepdims=True)
        return (y - mu) * lax.rsqrt(var + 1e-5) * g[...] + b[...]

    invE = 1.0 / jnp.maximum(dgv[0][:, :1], 1.0)
    invO = 1.0 / jnp.maximum(dgv[1][:, :1], 1.0)
    out[0] = half(p0[:, :DH], p1[:, :DH], invE, hE[...][0])
    out[1] = half(p0[:, DH:], p1[:, DH:], invO, hO[...][0])


def _tc_sage(p0r, p1r, degEO, hEO, wlT0, wlT1, bl, wrT, g, b):
    full = lambda i: (0, 0)
    core0 = lambda i: (0, i, 0)
    core1 = lambda i: (1, i, 0)
    return pl.pallas_call(
        _sage_post_body,
        grid=(N // BN,),
        in_specs=[
            pl.BlockSpec((1, BN // 2, D), core0),
            pl.BlockSpec((1, BN // 2, D), core1),
            pl.BlockSpec((1, BN // 2, D), core0),
            pl.BlockSpec((1, BN // 2, D), core1),
            pl.BlockSpec((2, BN // 2, 16), lambda i: (0, i, 0)),
            pl.BlockSpec((1, BN // 2, D), core0),
            pl.BlockSpec((1, BN // 2, D), core1),
            pl.BlockSpec((DH, D), full),
            pl.BlockSpec((DH, D), full),
            pl.BlockSpec((1, D), full),
            pl.BlockSpec((D, D), full),
            pl.BlockSpec((1, D), full),
            pl.BlockSpec((1, D), full),
        ],
        out_specs=pl.BlockSpec((2, BN // 2, D), lambda i: (0, i, 0)),
        out_shape=jax.ShapeDtypeStruct((2, N // 2, D), jnp.float32),
    )(p0r, p0r, p1r, p1r, degEO, hEO, hEO, wlT0, wlT1, bl, wrT, g, b)


def _lstm_mlp_body(e0, e1, e2, e3, wi0, wh0, b0, wi1, wh1, b1,
                   we1, be1, we2, be2, wd1, bd1, wd2, bd2, hm_out, hr_out):
    def layer(xs, wiT, whT, bb):
        hprev = jnp.zeros((BN, H), jnp.float32)
        c = jnp.zeros((BN, H), jnp.float32)
        hs = []
        for xt in xs:
            gates = (jnp.dot(xt, wiT, preferred_element_type=jnp.float32)
                     + jnp.dot(hprev, whT, preferred_element_type=jnp.float32)
                     + bb)
            i = jax.nn.sigmoid(gates[:, 0:H])
            f = jax.nn.sigmoid(gates[:, H:2 * H])
            gg = jnp.tanh(gates[:, 2 * H:3 * H])
            o = jax.nn.sigmoid(gates[:, 3 * H:4 * H])
            c = f * c + i * gg
            hprev = o * jnp.tanh(c)
            hs.append(hprev)
        return hs

    xs0 = [e0[...], e1[...], e2[...], e3[...]]
    hs0 = layer(xs0, wi0[...], wh0[...], b0[...])
    hs1 = layer(hs0, wi1[...], wh1[...], b1[...])
    hm = hs1[-1]
    z = jnp.maximum(
        jnp.dot(hm, we1[...], preferred_element_type=jnp.float32) + be1[...],
        0.0)
    z = jnp.dot(z, we2[...], preferred_element_type=jnp.float32) + be2[...]
    hr = jnp.maximum(
        jnp.dot(z, wd1[...], preferred_element_type=jnp.float32) + bd1[...],
        0.0)
    hr = jnp.dot(hr, wd2[...], preferred_element_type=jnp.float32) + bd2[...]
    hm_out[...] = hm
    hr_out[...] = hr


def _tc_lstm_mlp(e0, e1, e2, e3, wi0, wh0, b0, wi1, wh1, b1,
                 we1, be1, we2, be2, wd1, bd1, wd2, bd2):
    row = lambda i: (i, 0)
    full = lambda i: (0, 0)
    ws = [
        pl.BlockSpec((D, 4 * H), full),
        pl.BlockSpec((H, 4 * H), full),
        pl.BlockSpec((1, 4 * H), full),
        pl.BlockSpec((H, 4 * H), full),
        pl.BlockSpec((H, 4 * H), full),
        pl.BlockSpec((1, 4 * H), full),
        pl.BlockSpec((H, AEH), full),
        pl.BlockSpec((1, AEH), full),
        pl.BlockSpec((AEH, LAT), full),
        pl.BlockSpec((1, LAT), full),
        pl.BlockSpec((LAT, AEH), full),
        pl.BlockSpec((1, AEH), full),
        pl.BlockSpec((AEH, H), full),
        pl.BlockSpec((1, H), full),
    ]
    return pl.pallas_call(
        _lstm_mlp_body,
        grid=(N // BN,),
        in_specs=[pl.BlockSpec((BN, D), row)] * 4 + ws,
        out_specs=(pl.BlockSpec((BN, H), row), pl.BlockSpec((BN, H), row)),
        out_shape=(jax.ShapeDtypeStruct((N, H), jnp.float32),
                   jax.ShapeDtypeStruct((N, H), jnp.float32)),
    )(e0, e1, e2, e3, wi0, wh0, b0, wi1, wh1, b1,
      we1, be1, we2, be2, wd1, bd1, wd2, bd2)


def kernel(x, edge_index, Wl1, bl1, Wr1, ln1g, ln1b, Wl2, bl2, Wr2, ln2g,
           ln2b, Wih0, Whh0, bih0, bhh0, Wih1, Whh1, bih1, bhh1, We1, be1,
           We2, be2, Wd1, bd1, Wd2, bd2):
    src = edge_index[:, 0, :]
    dst = edge_index[:, 1, :]
    pad_i = jnp.arange(PADE, dtype=jnp.int32)
    pad_src = jnp.broadcast_to(pad_i % N, (T, PADE))
    pad_dst = jnp.broadcast_to(N + pad_i % (N_PAD - N), (T, PADE))
    srcf = jnp.concatenate([src, pad_src], axis=1)
    src2 = 2 * srcf
    srcp0 = src2.reshape(T, NW, NCHUNK, CH)
    srcp1 = (src2 + 1).reshape(T, NW, NCHUNK, CH)
    srcq = 2 * ((srcf % 2) * (N // 2) + srcf // 2)
    srcq0 = srcq.reshape(T, NW, NCHUNK, CH)
    srcq1 = (srcq + 1).reshape(T, NW, NCHUNK, CH)
    dstp = jnp.concatenate([dst, pad_dst], axis=1).reshape(T, NW, NCHUNK, CH)

    r2 = lambda v: v.reshape(1, -1)
    Wl1T, Wr1T, Wl2T, Wr2T = Wl1.T, Wr1.T, Wl2.T, Wr2.T
    Wl1T0, Wl1T1 = Wl1T[:DH], Wl1T[DH:]
    Wl2T0, Wl2T1 = Wl2T[:DH], Wl2T[DH:]
    b0 = r2(bih0 + bhh0)
    b1 = r2(bih1 + bhh1)

    zz64 = jnp.zeros((N_PAD, DH), jnp.float32)
    zz16 = jnp.zeros((N_PAD, 16), jnp.float32)

    l1 = []
    for t in range(T):
        ht2 = x[t].reshape(2 * N, DH)
        l1.append(_sc_agg_deg(ht2, srcp0[t], srcp1[t], dstp[t], zz64, zz16))

    xEO = x.reshape(T, N // 2, 2, D).swapaxes(1, 2)

    rs = lambda p: p.reshape(NC, N_PAD // 2, D)
    embeds = []
    for t in range(T):
        p0, p1, dgp = l1[t]
        dg = dgp[0, :N, :] + dgp[1, :N, :]
        degEO = dg.reshape(N // 2, 2, 16).swapaxes(0, 1)
        h1 = _tc_sage(rs(p0), rs(p1), degEO, xEO[t],
                      Wl1T0, Wl1T1, r2(bl1), Wr1T, r2(ln1g), r2(ln1b))
        q0, q1 = _sc_agg(h1.reshape(2 * N, DH), srcq0[t], srcq1[t], dstp[t],
                         zz64)
        h2 = _tc_sage(rs(q0), rs(q1), degEO, h1,
                      Wl2T0, Wl2T1, r2(bl2), Wr2T, r2(ln2g), r2(ln2b))
        embeds.append(h2.reshape(N, D))

    hm, hr = _tc_lstm_mlp(
        embeds[0], embeds[1], embeds[2], embeds[3],
        Wih0.T, Whh0.T, b0, Wih1.T, Whh1.T, b1,
        We1.T, r2(be1), We2.T, r2(be2), Wd1.T, r2(bd1), Wd2.T, r2(bd2))
    unperm = lambda v: v.reshape(2, N // 2, D).swapaxes(0, 1).reshape(N, D)
    return (unperm(hm), unperm(hr))
